# Initial kernel scaffold; baseline (speedup 1.0000x reference)
import jax
import jax.numpy as jnp
from jax.experimental import pallas as pl


def kernel(exercises, exercise_table, position_table):
    raise NotImplementedError("write your pallas kernel here")


# speedup vs baseline: 3.7012x; 3.7012x over previous
import functools

import jax
import jax.numpy as jnp
from jax import lax
from jax.experimental import pallas as pl
from jax.experimental.pallas import tpu as pltpu
from jax.experimental.pallas import tpu_sc as plsc

_B = 4096
_S = 200
_D = 64
_N = _B * _S

_NC = 2
_NS = 16
_NW = _NC * _NS
_PER_W = _N // _NW

_K = 800
_M = _K // _S
_SUB = 80
_NSUB = _K // _SUB
_CHUNKS = _PER_W // _K
_LANES = 16
_DV = _D // _LANES


def _body(idx_hbm, table_hbm, pos_hbm, out_hbm, idx_v, rows_v, pos_v, sem):
    cid = lax.axis_index("c")
    sid = lax.axis_index("s")
    wid = sid * _NC + cid

    pltpu.sync_copy(pos_hbm, pos_v)

    def chunk_body(ci, carry):
        base = wid * _PER_W + ci * _K
        pltpu.sync_copy(idx_hbm.at[pl.ds(base, _K)], idx_v)
        handles = []
        for j in range(_NSUB):
            handles.append(
                pltpu.async_copy(table_hbm.at[idx_v.at[pl.ds(j * _SUB, _SUB)]],
                                 rows_v.at[pl.ds(j * _SUB, _SUB)], sem))
        for h in handles:
            h.wait()

        def pos_body(s, c2):
            for d in range(_DV):
                p = pos_v[s, pl.ds(d * _LANES, _LANES)]
                for r in range(_M):
                    row = s + r * _S
                    rows_v[row, pl.ds(d * _LANES, _LANES)] = (
                        rows_v[row, pl.ds(d * _LANES, _LANES)] + p)
            return c2

        lax.fori_loop(0, _S, pos_body, 0, unroll=False)

        pltpu.sync_copy(rows_v, out_hbm.at[pl.ds(base, _K)])
        return carry

    lax.fori_loop(0, _CHUNKS, chunk_body, 0, unroll=False)


@jax.jit
def _embed(idx2d, table, pos):
    mesh = plsc.VectorSubcoreMesh(core_axis_name="c", subcore_axis_name="s")
    return pl.kernel(
        _body,
        out_type=jax.ShapeDtypeStruct((_N, _D), jnp.float32),
        mesh=mesh,
        compiler_params=pltpu.CompilerParams(use_tc_tiling_on_sc=False),
        scratch_types=[
            pltpu.VMEM((_K,), jnp.int32),
            pltpu.VMEM((_K, _D), jnp.float32),
            pltpu.VMEM((_S, _D), jnp.float32),
            pltpu.SemaphoreType.DMA,
        ],
    )(idx2d, table, pos)


def kernel(exercises, exercise_table, position_table):
    idx_flat = exercises.astype(jnp.int32).reshape(_N)
    out = _embed(idx_flat, exercise_table, position_table)
    return out.reshape(_B, _S, _D)


# speedup vs baseline: 4.2257x; 1.1417x over previous
import functools

import jax
import jax.numpy as jnp
from jax import lax
from jax.experimental import pallas as pl
from jax.experimental.pallas import tpu as pltpu
from jax.experimental.pallas import tpu_sc as plsc

_B = 4096
_S = 200
_D = 64
_N = _B * _S

_NC = 2
_NS = 16
_NW = _NC * _NS
_PER_W = _N // _NW

_K = 800
_M = _K // _S
_SUB = 80
_NSUB = _K // _SUB
_CHUNKS = _PER_W // _K
_LANES = 16
_DV = _D // _LANES


def _body(idx_hbm, table_hbm, pos_hbm, out_hbm,
          idx0, idx1, rows0, rows1, pos_v,
          gsem0, gsem1, ssem0, ssem1, isem0, isem1):
    cid = lax.axis_index("c")
    sid = lax.axis_index("s")
    wid = sid * _NC + cid

    def idx_slice(ci):
        return idx_hbm.at[pl.ds(wid * _PER_W + ci * _K, _K)]

    def out_slice(ci):
        return out_hbm.at[pl.ds(wid * _PER_W + ci * _K, _K)]

    def fire_gathers(idx_v, rows_v, sem):
        for j in range(_NSUB):
            pltpu.async_copy(table_hbm.at[idx_v.at[pl.ds(j * _SUB, _SUB)]],
                             rows_v.at[pl.ds(j * _SUB, _SUB)], sem)

    def wait_gathers(rows_v, sem):
        pltpu.make_async_copy(out_hbm.at[pl.ds(0, _K)], rows_v, sem).wait()

    def wait_idx(idx_v, sem):
        pltpu.make_async_copy(idx_hbm.at[pl.ds(0, _K)], idx_v, sem).wait()

    def wait_store(rows_v, sem):
        pltpu.make_async_copy(rows_v, out_hbm.at[pl.ds(0, _K)], sem).wait()

    def add_pos(rows_v):
        @plsc.parallel_loop(0, _S, 1, unroll=2)
        def _(s):
            for d in range(_DV):
                p = pos_v[s, pl.ds(d * _LANES, _LANES)]
                for r in range(_M):
                    row = s + r * _S
                    rows_v[row, pl.ds(d * _LANES, _LANES)] = (
                        rows_v[row, pl.ds(d * _LANES, _LANES)] + p)

    pltpu.sync_copy(pos_hbm, pos_v)
    pltpu.sync_copy(idx_slice(0), idx0)
    fire_gathers(idx0, rows0, gsem0)
    pltpu.async_copy(idx_slice(1), idx1, isem1)

    T = _CHUNKS // 2

    def super_body(t, carry):
        a = 2 * t
        b = a + 1

        @pl.when(t > 0)
        def _():
            wait_store(rows1, ssem1)
        wait_idx(idx1, isem1)
        fire_gathers(idx1, rows1, gsem1)

        wait_gathers(rows0, gsem0)
        @pl.when(t < T - 1)
        def _():
            pltpu.async_copy(idx_slice(a + 2), idx0, isem0)
        add_pos(rows0)
        pltpu.async_copy(rows0, out_slice(a), ssem0)
        @pl.when(t < T - 1)
        def _():
            wait_idx(idx0, isem0)
            wait_store(rows0, ssem0)
            fire_gathers(idx0, rows0, gsem0)

        wait_gathers(rows1, gsem1)
        @pl.when(t < T - 1)
        def _():
            pltpu.async_copy(idx_slice(b + 2), idx1, isem1)
        add_pos(rows1)
        pltpu.async_copy(rows1, out_slice(b), ssem1)
        return carry

    lax.fori_loop(0, T, super_body, 0, unroll=False)

    wait_store(rows0, ssem0)
    wait_store(rows1, ssem1)


@jax.jit
def _embed(idx_flat, table, pos):
    mesh = plsc.VectorSubcoreMesh(core_axis_name="c", subcore_axis_name="s")
    return pl.kernel(
        _body,
        out_type=jax.ShapeDtypeStruct((_N, _D), jnp.float32),
        mesh=mesh,
        compiler_params=pltpu.CompilerParams(use_tc_tiling_on_sc=False),
        scratch_types=[
            pltpu.VMEM((_K,), jnp.int32),
            pltpu.VMEM((_K,), jnp.int32),
            pltpu.VMEM((_K, _D), jnp.float32),
            pltpu.VMEM((_K, _D), jnp.float32),
            pltpu.VMEM((_S, _D), jnp.float32),
            pltpu.SemaphoreType.DMA,
            pltpu.SemaphoreType.DMA,
            pltpu.SemaphoreType.DMA,
            pltpu.SemaphoreType.DMA,
            pltpu.SemaphoreType.DMA,
            pltpu.SemaphoreType.DMA,
        ],
    )(idx_flat, table, pos)


def kernel(exercises, exercise_table, position_table):
    idx_flat = exercises.astype(jnp.int32).reshape(_N)
    out = _embed(idx_flat, exercise_table, position_table)
    return out.reshape(_B, _S, _D)
